# baseline (speedup 1.0000x reference)
import functools

import jax
import jax.numpy as jnp
from jax.experimental import pallas as pl
from jax.experimental.pallas import tpu as pltpu


def _lstm_fc_kernel(x_hbm, h0_ref, c0_ref, wih1_ref, wcomb_ref, bias_ref,
                    fcw_ref, fcb_ref,
                    out_ref, hN_ref, cN_ref,
                    x_vmem, sem,
                    *, T):
    L, B, H = h0_ref.shape
    G = 4 * H
    I = wih1_ref.shape[0]

    copies = [
        pltpu.make_async_copy(x_hbm.at[:, k, :], x_vmem.at[k], sem.at[k])
        for k in range(T)
    ]
    for c in copies:
        c.start()

    lane = jax.lax.broadcasted_iota(jnp.int32, (1, 4 * H), 1)
    not_g = (lane // H) != 2
    wsc = jnp.where(jnp.concatenate([not_g] * L, axis=1), 0.5, 1.0)
    wcomb = (wcomb_ref[...] * wsc).astype(jnp.bfloat16)
    fcw = fcw_ref[...].astype(jnp.bfloat16)
    bias = bias_ref[...] * wsc
    wih1 = (wih1_ref[...] * wsc[:, 0:G]).astype(jnp.bfloat16)
    bias0 = bias[:, 0:G]

    h_st = [h0_ref[l].astype(jnp.bfloat16) for l in range(L)]
    hf_st = [h0_ref[l] for l in range(L)]
    c_st = [c0_ref[l] for l in range(L)]

    def update(l, zl):
        y = jnp.tanh(zl)
        c_new = ((y[:, H:2 * H] + 1.0) * c_st[l]
                 + (y[:, 0:H] + 1.0) * y[:, 2 * H:3 * H]) * 0.5
        h_new = (y[:, 3 * H:4 * H] + 1.0) * jnp.tanh(c_new) * 0.5
        c_st[l] = c_new
        hf_st[l] = h_new
        h_st[l] = h_new.astype(jnp.bfloat16)

    for r in range(T + L - 1):
        z = jnp.dot(jnp.concatenate(h_st, axis=1), wcomb,
                    preferred_element_type=jnp.float32)
        if r < T:
            if r == 0:
                for c in copies:
                    c.wait()
            pre = jnp.dot(x_vmem[r].astype(jnp.bfloat16), wih1,
                          preferred_element_type=jnp.float32)
            update(0, z[:, 0:G] + pre + bias0)
        t1 = r - (L - 1)
        if 0 <= t1 < T:
            update(1, z[:, G:2 * G] + bias[:, G:2 * G])
            o_fc = jnp.dot(h_st[1], fcw, preferred_element_type=jnp.float32)
            out_ref[:, t1, :] = o_fc + fcb_ref[...]

    for l in range(L):
        hN_ref[l] = hf_st[l]
        cN_ref[l] = c_st[l]


@jax.jit
def kernel(x, h0, c0, wih1_t, wcomb, bias, fc_w, fc_b):
    B, T, I = x.shape
    L, _, H = h0.shape
    O_pad = fc_w.shape[-1]
    O = 128

    kern = functools.partial(_lstm_fc_kernel, T=T)

    out3, hN, cN = pl.pallas_call(
        kern,
        out_shape=(jax.ShapeDtypeStruct((B, T, O_pad), jnp.float32),
                   jax.ShapeDtypeStruct((L, B, H), jnp.float32),
                   jax.ShapeDtypeStruct((L, B, H), jnp.float32)),
        grid=(1,),
        in_specs=[
            pl.BlockSpec(memory_space=pl.ANY),
            pl.BlockSpec((L, B, H), lambda j: (0, 0, 0)),
            pl.BlockSpec((L, B, H), lambda j: (0, 0, 0)),
            pl.BlockSpec(wih1_t.shape, lambda j: (0, 0)),
            pl.BlockSpec(wcomb.shape, lambda j: (0, 0)),
            pl.BlockSpec(bias.shape, lambda j: (0, 0)),
            pl.BlockSpec(fc_w.shape, lambda j: (0, 0)),
            pl.BlockSpec(fc_b.shape, lambda j: (0, 0)),
        ],
        out_specs=[
            pl.BlockSpec((B, T, O_pad), lambda j: (0, 0, 0)),
            pl.BlockSpec((L, B, H), lambda j: (0, 0, 0)),
            pl.BlockSpec((L, B, H), lambda j: (0, 0, 0)),
        ],
        scratch_shapes=[
            pltpu.VMEM((T, B, I), jnp.float32),
            pltpu.SemaphoreType.DMA((T,)),
        ],
        compiler_params=pltpu.CompilerParams(
            dimension_semantics=("arbitrary",)),
    )(x, h0, c0, wih1_t, wcomb, bias, fc_w, fc_b)

    return out3.reshape(B * T, O), (hN, cN)


# speedup vs baseline: 1.0802x; 1.0802x over previous
import functools

import jax
import jax.numpy as jnp
from jax.experimental import pallas as pl
from jax.experimental.pallas import tpu as pltpu


def _lstm_fc_kernel(x_hbm, h0_ref, c0_ref, wih1_ref, wcomb_ref, bias_ref,
                    fcw_ref, fcb_ref,
                    out_ref, hN_ref, cN_ref,
                    x_vmem, sem,
                    *, T):
    L, B, H = h0_ref.shape
    G = 4 * H
    I = wih1_ref.shape[0]

    copies = [
        pltpu.make_async_copy(x_hbm.at[:, k, :], x_vmem.at[k], sem.at[k])
        for k in range(T)
    ]
    for c in copies:
        c.start()

    lane = jax.lax.broadcasted_iota(jnp.int32, (1, 4 * H), 1)
    not_g = (lane // H) != 2
    wsc = jnp.where(jnp.concatenate([not_g] * L, axis=1), 0.5, 1.0)
    wcomb = (wcomb_ref[...] * wsc).astype(jnp.bfloat16)
    fcw = fcw_ref[...].astype(jnp.bfloat16)
    bias = bias_ref[...] * wsc
    wih1 = (wih1_ref[...] * wsc[:, 0:G]).astype(jnp.bfloat16)
    bias0 = bias[:, 0:G]

    h_st = [h0_ref[l].astype(jnp.bfloat16) for l in range(L)]
    hf_st = [h0_ref[l] for l in range(L)]
    c_st = [c0_ref[l] for l in range(L)]

    def update(l, zl):
        y = jnp.tanh(zl)
        c_new = ((y[:, H:2 * H] + 1.0) * c_st[l]
                 + (y[:, 0:H] + 1.0) * y[:, 2 * H:3 * H]) * 0.5
        h_new = (y[:, 3 * H:4 * H] + 1.0) * jnp.tanh(c_new) * 0.5
        c_st[l] = c_new
        hf_st[l] = h_new
        h_st[l] = h_new.astype(jnp.bfloat16)

    for r in range(T + L - 1):
        z = jnp.dot(jnp.concatenate(h_st, axis=1), wcomb,
                    preferred_element_type=jnp.float32)
        if r < T:
            copies[r].wait()
            pre = jnp.dot(x_vmem[r].astype(jnp.bfloat16), wih1,
                          preferred_element_type=jnp.float32)
            update(0, z[:, 0:G] + pre + bias0)
        t1 = r - (L - 1)
        if 0 <= t1 < T:
            update(1, z[:, G:2 * G] + bias[:, G:2 * G])
            o_fc = jnp.dot(h_st[1], fcw, preferred_element_type=jnp.float32)
            out_ref[:, t1, :] = o_fc + fcb_ref[...]

    for l in range(L):
        hN_ref[l] = hf_st[l]
        cN_ref[l] = c_st[l]


@jax.jit
def kernel(x, h0, c0, wih1_t, wcomb, bias, fc_w, fc_b):
    B, T, I = x.shape
    L, _, H = h0.shape
    O_pad = fc_w.shape[-1]
    O = 128

    kern = functools.partial(_lstm_fc_kernel, T=T)

    out3, hN, cN = pl.pallas_call(
        kern,
        out_shape=(jax.ShapeDtypeStruct((B, T, O_pad), jnp.float32),
                   jax.ShapeDtypeStruct((L, B, H), jnp.float32),
                   jax.ShapeDtypeStruct((L, B, H), jnp.float32)),
        grid=(1,),
        in_specs=[
            pl.BlockSpec(memory_space=pl.ANY),
            pl.BlockSpec((L, B, H), lambda j: (0, 0, 0)),
            pl.BlockSpec((L, B, H), lambda j: (0, 0, 0)),
            pl.BlockSpec(wih1_t.shape, lambda j: (0, 0)),
            pl.BlockSpec(wcomb.shape, lambda j: (0, 0)),
            pl.BlockSpec(bias.shape, lambda j: (0, 0)),
            pl.BlockSpec(fc_w.shape, lambda j: (0, 0)),
            pl.BlockSpec(fc_b.shape, lambda j: (0, 0)),
        ],
        out_specs=[
            pl.BlockSpec((B, T, O_pad), lambda j: (0, 0, 0)),
            pl.BlockSpec((L, B, H), lambda j: (0, 0, 0)),
            pl.BlockSpec((L, B, H), lambda j: (0, 0, 0)),
        ],
        scratch_shapes=[
            pltpu.VMEM((T, B, I), jnp.float32),
            pltpu.SemaphoreType.DMA((T,)),
        ],
        compiler_params=pltpu.CompilerParams(
            dimension_semantics=("arbitrary",)),
    )(x, h0, c0, wih1_t, wcomb, bias, fc_w, fc_b)

    return out3.reshape(B * T, O), (hN, cN)
